# baseline (speedup 1.0000x reference)
import functools

import jax
import jax.numpy as jnp
from jax import lax
from jax.experimental import pallas as pl
from jax.experimental.pallas import tpu as pltpu
from jax.experimental.pallas import tpu_sc as plsc

N = 10000
NP = 10240
E = 160000
D_IN = 256
F = 16
NCLS = 7

NC = 2
NS = 16
L = 16
NW = NC * NS
RPT = NP // NS
BM = 1024


def _sc_mesh():
    return plsc.VectorSubcoreMesh(core_axis_name="c", subcore_axis_name="s",
                                  num_cores=NC, num_subcores=NS)


_SC_PARAMS = pltpu.CompilerParams(needs_layout_passes=False,
                                  use_tc_tiling_on_sc=False)


KE16 = E // L


def _make_aggregate(FW, CHG):
    G = NS // FW
    SPLIT = NC * G
    KEQ = KE16 // SPLIT
    NCH = KEQ // CHG
    assert KEQ % CHG == 0 and NS % FW == 0 and KE16 % SPLIT == 0

    @functools.partial(
        pl.kernel,
        mesh=_sc_mesh(),
        compiler_params=_SC_PARAMS,
        out_type=jax.ShapeDtypeStruct((SPLIT, FW, NP), jnp.float32),
        scratch_types=[
            pltpu.VMEM((2, CHG, L), jnp.int32),
            pltpu.VMEM((2, CHG, L), jnp.int32),
            pltpu.VMEM((NP,), jnp.float32),
            pltpu.VMEM((NP,), jnp.float32),
            pltpu.SemaphoreType.DMA,
            pltpu.SemaphoreType.DMA,
            pltpu.SemaphoreType.DMA,
            pltpu.SemaphoreType.DMA,
        ],
    )
    def k(u_hbm, src_hbm, dst_hbm, out_hbm, src_v, dst_v, u_v, acc_v,
          sem_s0, sem_s1, sem_d0, sem_d1):
        c = lax.axis_index("c")
        s = lax.axis_index("s")
        col = s % FW
        q = c * G + s // FW

        ssems = (sem_s0, sem_s1)
        dsems = (sem_d0, sem_d1)

        def start(t, slot):
            pltpu.async_copy(src_hbm.at[q, pl.ds(t * CHG, CHG)],
                             src_v.at[slot], ssems[slot])
            pltpu.async_copy(dst_hbm.at[q, pl.ds(t * CHG, CHG)],
                             dst_v.at[slot], dsems[slot])

        def wait(t, slot):
            pltpu.make_async_copy(src_hbm.at[q, pl.ds(t * CHG, CHG)],
                                  src_v.at[slot], ssems[slot]).wait()
            pltpu.make_async_copy(dst_hbm.at[q, pl.ds(t * CHG, CHG)],
                                  dst_v.at[slot], dsems[slot]).wait()

        start(0, 0)
        pltpu.sync_copy(u_hbm.at[col], u_v)

        z16 = jnp.zeros((L,), jnp.float32)

        @plsc.parallel_loop(0, NP // L, unroll=4)
        def _(i):
            acc_v[pl.ds(i * L, L)] = z16

        for t in range(NCH):
            slot = t % 2
            if t + 1 < NCH:
                start(t + 1, (t + 1) % 2)
            wait(t, slot)

            @plsc.parallel_loop(0, CHG, unroll=4)
            def _(gf):
                sidx = src_v[slot, gf, :]
                didx = dst_v[slot, gf, :]
                vals = plsc.load_gather(u_v, [sidx])
                plsc.addupdate_scatter(acc_v, [didx], vals)

        pltpu.sync_copy(acc_v, out_hbm.at[q, col])

    return k


_agg8 = _make_aggregate(8, 500)

GPC = KE16 // NS
CH1 = 500
NCH1 = (KE16 // NC) // CH1


def _agg16_deg(uT, src2, dst2):
    @functools.partial(
        pl.kernel,
        mesh=_sc_mesh(),
        compiler_params=_SC_PARAMS,
        out_type=[
            jax.ShapeDtypeStruct((NC, F, NP), jnp.float32),
            jax.ShapeDtypeStruct((1, NP), jnp.float32),
        ],
        scratch_types=[
            pltpu.VMEM((2, CH1, L), jnp.int32),
            pltpu.VMEM((2, CH1, L), jnp.int32),
            pltpu.VMEM((GPC, L), jnp.int32),
            pltpu.VMEM((NP,), jnp.float32),
            pltpu.VMEM((NS, RPT), jnp.float32),
            pltpu.VMEM((NP,), jnp.float32),
            pltpu.VMEM((NP,), jnp.float32),
            pltpu.VMEM((NP,), jnp.float32),
            pltpu.VMEM_SHARED((NS, NP), jnp.float32),
            pltpu.VMEM_SHARED((NP,), jnp.float32),
            pltpu.SemaphoreType.DMA,
            pltpu.SemaphoreType.DMA,
            pltpu.SemaphoreType.DMA,
            pltpu.SemaphoreType.DMA,
            pltpu.SemaphoreType.DMA,
            pltpu.SemaphoreType.DMA,
        ],
    )
    def k(u_hbm, src_hbm, dst_hbm, out_hbm, dinv_hbm,
          src_v, dst_v, hist_v, deg_v, red_v, dinv_v, u_v, acc_v,
          deg_sh, dinv_sh, sem_s0, sem_s1, sem_d0, sem_d1, sem_h, sem_u):
        c = lax.axis_index("c")
        s = lax.axis_index("s")

        ssems = (sem_s0, sem_s1)
        dsems = (sem_d0, sem_d1)

        def start(t, slot):
            pltpu.async_copy(src_hbm.at[c, pl.ds(t * CH1, CH1)],
                             src_v.at[slot], ssems[slot])
            pltpu.async_copy(dst_hbm.at[c, pl.ds(t * CH1, CH1)],
                             dst_v.at[slot], dsems[slot])

        def wait(t, slot):
            pltpu.make_async_copy(src_hbm.at[c, pl.ds(t * CH1, CH1)],
                                  src_v.at[slot], ssems[slot]).wait()
            pltpu.make_async_copy(dst_hbm.at[c, pl.ds(t * CH1, CH1)],
                                  dst_v.at[slot], dsems[slot]).wait()

        hc = s // (NS // NC)
        hr = s % (NS // NC)
        pltpu.async_copy(dst_hbm.at[hc, pl.ds(hr * GPC, GPC)], hist_v, sem_h)
        pltpu.async_copy(u_hbm.at[s], u_v, sem_u)
        start(0, 0)

        z16 = jnp.zeros((L,), jnp.float32)

        @plsc.parallel_loop(0, NP // L, unroll=4)
        def _(i):
            deg_v[pl.ds(i * L, L)] = z16
            acc_v[pl.ds(i * L, L)] = z16

        pltpu.make_async_copy(dst_hbm.at[hc, pl.ds(hr * GPC, GPC)],
                              hist_v, sem_h).wait()
        ones16 = jnp.ones((L,), jnp.float32)

        @plsc.parallel_loop(0, GPC, unroll=4)
        def _(g):
            plsc.addupdate_scatter(deg_v, [hist_v[g, :]], ones16)

        pltpu.sync_copy(deg_v, deg_sh.at[s])
        plsc.subcore_barrier()

        for r in range(NS):
            pltpu.sync_copy(deg_sh.at[r, pl.ds(s * RPT, RPT)], red_v.at[r])

        half = jnp.full((L,), 0.5, jnp.float32)
        three_half = jnp.full((L,), 1.5, jnp.float32)
        magic = jnp.full((L,), 0x5F3759DF, jnp.int32)

        @plsc.parallel_loop(0, RPT // L, unroll=2)
        def _(t):
            d = red_v[0, pl.ds(t * L, L)]
            for r in range(1, NS):
                d = d + red_v[r, pl.ds(t * L, L)]
            d = d + 1.0
            yi = magic - lax.shift_right_arithmetic(plsc.bitcast(d, jnp.int32),
                                                    jnp.full((L,), 1, jnp.int32))
            y = plsc.bitcast(yi, jnp.float32)
            hd = d * half
            for _ in range(3):
                y = y * (three_half - hd * y * y)
            deg_v[pl.ds(t * L, L)] = y

        pltpu.sync_copy(deg_v.at[pl.ds(0, RPT)],
                        dinv_sh.at[pl.ds(s * RPT, RPT)])

        @pl.when(c == 0)
        def _():
            pltpu.sync_copy(deg_v.at[pl.ds(0, RPT)],
                            dinv_hbm.at[0, pl.ds(s * RPT, RPT)])

        plsc.subcore_barrier()
        pltpu.sync_copy(dinv_sh, dinv_v)
        pltpu.make_async_copy(u_hbm.at[s], u_v, sem_u).wait()

        for t in range(NCH1):
            slot = t % 2
            if t + 1 < NCH1:
                start(t + 1, (t + 1) % 2)
            wait(t, slot)

            @plsc.parallel_loop(0, CH1, unroll=4)
            def _(gf):
                sidx = src_v[slot, gf, :]
                didx = dst_v[slot, gf, :]
                vals = plsc.load_gather(u_v, [sidx]) * plsc.load_gather(dinv_v, [sidx])
                plsc.addupdate_scatter(acc_v, [didx], vals)

        pltpu.sync_copy(acc_v, out_hbm.at[c, s])

    return k(uT, src2, dst2)


def _tc1(x, W1):
    def body(x_ref, w_ref, u_ref):
        u_ref[...] = lax.dot_general(w_ref[...], x_ref[...],
                                     (((0,), (1,)), ((), ())),
                                     preferred_element_type=jnp.float32)

    return pl.pallas_call(
        body,
        grid=(NP // BM,),
        in_specs=[
            pl.BlockSpec((BM, D_IN), lambda i: (i, 0)),
            pl.BlockSpec((D_IN, F), lambda i: (0, 0)),
        ],
        out_specs=pl.BlockSpec((F, BM), lambda i: (0, i)),
        out_shape=jax.ShapeDtypeStruct((F, NP), jnp.float32),
    )(x, W1)


F8 = 8


def _tc2(u1T, p1, dinvT, b1c, W2):
    def body(u_ref, p_ref, dinv_ref, b_ref, w_ref, u2_ref):
        tot = u_ref[...] * dinv_ref[...] + p_ref[0] + p_ref[1]
        h = jnp.maximum(tot * dinv_ref[...] + b_ref[...], 0.0)
        u2 = lax.dot_general(w_ref[...], h, (((0,), (0,)), ((), ())),
                             preferred_element_type=jnp.float32)
        u2_ref[0:NCLS, :] = u2 * dinv_ref[...]
        u2_ref[NCLS:F8, :] = jnp.zeros((F8 - NCLS, BM), jnp.float32)

    return pl.pallas_call(
        body,
        grid=(NP // BM,),
        in_specs=[
            pl.BlockSpec((F, BM), lambda i: (0, i)),
            pl.BlockSpec((NC, F, BM), lambda i: (0, 0, i)),
            pl.BlockSpec((1, BM), lambda i: (0, i)),
            pl.BlockSpec((F, 1), lambda i: (0, 0)),
            pl.BlockSpec((F, NCLS), lambda i: (0, 0)),
        ],
        out_specs=pl.BlockSpec((F8, BM), lambda i: (0, i)),
        out_shape=jax.ShapeDtypeStruct((F8, NP), jnp.float32),
    )(u1T, p1, dinvT, b1c, W2)


def _tc3(u2T, p2, dinvT, b2c):
    def body(u_ref, p_ref, dinv_ref, b_ref, o_ref):
        tot = (u_ref[...] + p_ref[0] + p_ref[1] + p_ref[2] + p_ref[3])
        tot = tot * dinv_ref[...]
        z = tot[:NCLS, :] + b_ref[...]
        m = jnp.max(z, axis=0, keepdims=True)
        lse = jnp.log(jnp.sum(jnp.exp(z - m), axis=0, keepdims=True)) + m
        o_ref[...] = (z - lse).T

    return pl.pallas_call(
        body,
        grid=(NP // BM,),
        in_specs=[
            pl.BlockSpec((F8, BM), lambda i: (0, i)),
            pl.BlockSpec((4, F8, BM), lambda i: (0, 0, i)),
            pl.BlockSpec((1, BM), lambda i: (0, i)),
            pl.BlockSpec((NCLS, 1), lambda i: (0, 0)),
        ],
        out_specs=pl.BlockSpec((BM, NCLS), lambda i: (i, 0)),
        out_shape=jax.ShapeDtypeStruct((N, NCLS), jnp.float32),
    )(u2T, p2, dinvT, b2c)


def kernel(x, edge_index, W1, b1, W2, b2):
    src = edge_index[0]
    dst = edge_index[1]
    srcQ1 = src.reshape(NC, KE16 // NC, L)
    dstQ1 = dst.reshape(NC, KE16 // NC, L)
    srcQ2 = src.reshape(4, KE16 // 4, L)
    dstQ2 = dst.reshape(4, KE16 // 4, L)

    u1T = _tc1(x, W1)
    p1, dinvT = _agg16_deg(u1T, srcQ1, dstQ1)
    u2T = _tc2(u1T, p1, dinvT, b1.reshape(F, 1), W2)
    p2 = _agg8(u2T, srcQ2, dstQ2)
    return _tc3(u2T, p2, dinvT, b2.reshape(NCLS, 1))


# speedup vs baseline: 1.0291x; 1.0291x over previous
import functools

import jax
import jax.numpy as jnp
from jax import lax
from jax.experimental import pallas as pl
from jax.experimental.pallas import tpu as pltpu
from jax.experimental.pallas import tpu_sc as plsc

N = 10000
NP = 10240
E = 160000
D_IN = 256
F = 16
NCLS = 7

NC = 2
NS = 16
L = 16
NW = NC * NS
RPT = NP // NS
BM = 1024


def _sc_mesh():
    return plsc.VectorSubcoreMesh(core_axis_name="c", subcore_axis_name="s",
                                  num_cores=NC, num_subcores=NS)


_SC_PARAMS = pltpu.CompilerParams(needs_layout_passes=False,
                                  use_tc_tiling_on_sc=False)


KE16 = E // L


def _make_aggregate(FW, CHG):
    G = NS // FW
    SPLIT = NC * G
    KEQ = KE16 // SPLIT
    NCH = KEQ // CHG
    assert KEQ % CHG == 0 and NS % FW == 0 and KE16 % SPLIT == 0

    @functools.partial(
        pl.kernel,
        mesh=_sc_mesh(),
        compiler_params=_SC_PARAMS,
        out_type=jax.ShapeDtypeStruct((SPLIT, FW, NP), jnp.float32),
        scratch_types=[
            pltpu.VMEM((2, CHG, L), jnp.int32),
            pltpu.VMEM((2, CHG, L), jnp.int32),
            pltpu.VMEM((NP,), jnp.float32),
            pltpu.VMEM((NP,), jnp.float32),
            pltpu.SemaphoreType.DMA,
            pltpu.SemaphoreType.DMA,
            pltpu.SemaphoreType.DMA,
            pltpu.SemaphoreType.DMA,
        ],
    )
    def k(u_hbm, src_hbm, dst_hbm, out_hbm, src_v, dst_v, u_v, acc_v,
          sem_s0, sem_s1, sem_d0, sem_d1):
        c = lax.axis_index("c")
        s = lax.axis_index("s")
        col = s % FW
        q = c * G + s // FW

        ssems = (sem_s0, sem_s1)
        dsems = (sem_d0, sem_d1)

        def start(t, slot):
            pltpu.async_copy(src_hbm.at[q, pl.ds(t * CHG, CHG)],
                             src_v.at[slot], ssems[slot])
            pltpu.async_copy(dst_hbm.at[q, pl.ds(t * CHG, CHG)],
                             dst_v.at[slot], dsems[slot])

        def wait(t, slot):
            pltpu.make_async_copy(src_hbm.at[q, pl.ds(t * CHG, CHG)],
                                  src_v.at[slot], ssems[slot]).wait()
            pltpu.make_async_copy(dst_hbm.at[q, pl.ds(t * CHG, CHG)],
                                  dst_v.at[slot], dsems[slot]).wait()

        start(0, 0)
        pltpu.sync_copy(u_hbm.at[col], u_v)

        z16 = jnp.zeros((L,), jnp.float32)

        @plsc.parallel_loop(0, NP // L, unroll=4)
        def _(i):
            acc_v[pl.ds(i * L, L)] = z16

        for t in range(NCH):
            slot = t % 2
            if t + 1 < NCH:
                start(t + 1, (t + 1) % 2)
            wait(t, slot)

            @plsc.parallel_loop(0, CHG, unroll=4)
            def _(gf):
                sidx = src_v[slot, gf, :]
                didx = dst_v[slot, gf, :]
                vals = plsc.load_gather(u_v, [sidx])
                plsc.addupdate_scatter(acc_v, [didx], vals)

        pltpu.sync_copy(acc_v, out_hbm.at[q, col])

    return k


_agg8 = _make_aggregate(8, 500)

GPC = KE16 // NS
CH1 = 500
NCH1 = (KE16 // NC) // CH1


def _agg16_deg(uT, src2, dst2):
    @functools.partial(
        pl.kernel,
        mesh=_sc_mesh(),
        compiler_params=_SC_PARAMS,
        out_type=[
            jax.ShapeDtypeStruct((NC, F, NP), jnp.float32),
            jax.ShapeDtypeStruct((1, NP), jnp.float32),
        ],
        scratch_types=[
            pltpu.VMEM((2, CH1, L), jnp.int32),
            pltpu.VMEM((2, CH1, L), jnp.int32),
            pltpu.VMEM((GPC, L), jnp.int32),
            pltpu.VMEM((NP,), jnp.float32),
            pltpu.VMEM((NS, RPT), jnp.float32),
            pltpu.VMEM((NP,), jnp.float32),
            pltpu.VMEM((NP,), jnp.float32),
            pltpu.VMEM((NP,), jnp.float32),
            pltpu.VMEM_SHARED((NS, NP), jnp.float32),
            pltpu.VMEM_SHARED((NP,), jnp.float32),
            pltpu.SemaphoreType.DMA,
            pltpu.SemaphoreType.DMA,
            pltpu.SemaphoreType.DMA,
            pltpu.SemaphoreType.DMA,
            pltpu.SemaphoreType.DMA,
            pltpu.SemaphoreType.DMA,
        ],
    )
    def k(u_hbm, src_hbm, dst_hbm, out_hbm, dinv_hbm,
          src_v, dst_v, hist_v, deg_v, red_v, dinv_v, u_v, acc_v,
          deg_sh, dinv_sh, sem_s0, sem_s1, sem_d0, sem_d1, sem_h, sem_u):
        c = lax.axis_index("c")
        s = lax.axis_index("s")

        ssems = (sem_s0, sem_s1)
        dsems = (sem_d0, sem_d1)

        def start(t, slot):
            pltpu.async_copy(src_hbm.at[c, pl.ds(t * CH1, CH1)],
                             src_v.at[slot], ssems[slot])
            pltpu.async_copy(dst_hbm.at[c, pl.ds(t * CH1, CH1)],
                             dst_v.at[slot], dsems[slot])

        def wait(t, slot):
            pltpu.make_async_copy(src_hbm.at[c, pl.ds(t * CH1, CH1)],
                                  src_v.at[slot], ssems[slot]).wait()
            pltpu.make_async_copy(dst_hbm.at[c, pl.ds(t * CH1, CH1)],
                                  dst_v.at[slot], dsems[slot]).wait()

        hc = s // (NS // NC)
        hr = s % (NS // NC)
        pltpu.async_copy(dst_hbm.at[hc, pl.ds(hr * GPC, GPC)], hist_v, sem_h)
        pltpu.async_copy(u_hbm.at[s], u_v, sem_u)
        start(0, 0)

        z16 = jnp.zeros((L,), jnp.float32)

        @plsc.parallel_loop(0, NP // L, unroll=4)
        def _(i):
            deg_v[pl.ds(i * L, L)] = z16
            acc_v[pl.ds(i * L, L)] = z16

        pltpu.make_async_copy(dst_hbm.at[hc, pl.ds(hr * GPC, GPC)],
                              hist_v, sem_h).wait()
        ones16 = jnp.ones((L,), jnp.float32)

        @plsc.parallel_loop(0, GPC, unroll=4)
        def _(g):
            plsc.addupdate_scatter(deg_v, [hist_v[g, :]], ones16)

        pltpu.sync_copy(deg_v, deg_sh.at[s])
        plsc.subcore_barrier()

        for r in range(NS):
            pltpu.sync_copy(deg_sh.at[r, pl.ds(s * RPT, RPT)], red_v.at[r])

        half = jnp.full((L,), 0.5, jnp.float32)
        three_half = jnp.full((L,), 1.5, jnp.float32)
        magic = jnp.full((L,), 0x5F3759DF, jnp.int32)

        @plsc.parallel_loop(0, RPT // L, unroll=2)
        def _(t):
            d = red_v[0, pl.ds(t * L, L)]
            for r in range(1, NS):
                d = d + red_v[r, pl.ds(t * L, L)]
            d = d + 1.0
            yi = magic - lax.shift_right_arithmetic(plsc.bitcast(d, jnp.int32),
                                                    jnp.full((L,), 1, jnp.int32))
            y = plsc.bitcast(yi, jnp.float32)
            hd = d * half
            for _ in range(3):
                y = y * (three_half - hd * y * y)
            deg_v[pl.ds(t * L, L)] = y

        pltpu.sync_copy(deg_v.at[pl.ds(0, RPT)],
                        dinv_sh.at[pl.ds(s * RPT, RPT)])

        @pl.when(c == 0)
        def _():
            pltpu.sync_copy(deg_v.at[pl.ds(0, RPT)],
                            dinv_hbm.at[0, pl.ds(s * RPT, RPT)])

        plsc.subcore_barrier()
        pltpu.sync_copy(dinv_sh, dinv_v)
        pltpu.make_async_copy(u_hbm.at[s], u_v, sem_u).wait()

        @plsc.parallel_loop(0, NP // L, unroll=4)
        def _(i):
            u_v[pl.ds(i * L, L)] = u_v[pl.ds(i * L, L)] * dinv_v[pl.ds(i * L, L)]

        for t in range(NCH1):
            slot = t % 2
            if t + 1 < NCH1:
                start(t + 1, (t + 1) % 2)
            wait(t, slot)

            @plsc.parallel_loop(0, CH1, unroll=4)
            def _(gf):
                sidx = src_v[slot, gf, :]
                didx = dst_v[slot, gf, :]
                vals = plsc.load_gather(u_v, [sidx])
                plsc.addupdate_scatter(acc_v, [didx], vals)

        pltpu.sync_copy(acc_v, out_hbm.at[c, s])

    return k(uT, src2, dst2)


def _tc1(x, W1):
    def body(x_ref, w_ref, u_ref):
        u_ref[...] = lax.dot_general(w_ref[...], x_ref[...],
                                     (((0,), (1,)), ((), ())),
                                     preferred_element_type=jnp.float32)

    return pl.pallas_call(
        body,
        grid=(NP // BM,),
        in_specs=[
            pl.BlockSpec((BM, D_IN), lambda i: (i, 0)),
            pl.BlockSpec((D_IN, F), lambda i: (0, 0)),
        ],
        out_specs=pl.BlockSpec((F, BM), lambda i: (0, i)),
        out_shape=jax.ShapeDtypeStruct((F, NP), jnp.float32),
    )(x, W1)


F8 = 8


def _tc2(u1T, p1, dinvT, b1c, W2):
    def body(u_ref, p_ref, dinv_ref, b_ref, w_ref, u2_ref):
        tot = u_ref[...] * dinv_ref[...] + p_ref[0] + p_ref[1]
        h = jnp.maximum(tot * dinv_ref[...] + b_ref[...], 0.0)
        u2 = lax.dot_general(w_ref[...], h, (((0,), (0,)), ((), ())),
                             preferred_element_type=jnp.float32)
        u2_ref[0:NCLS, :] = u2 * dinv_ref[...]
        u2_ref[NCLS:F8, :] = jnp.zeros((F8 - NCLS, BM), jnp.float32)

    return pl.pallas_call(
        body,
        grid=(NP // BM,),
        in_specs=[
            pl.BlockSpec((F, BM), lambda i: (0, i)),
            pl.BlockSpec((NC, F, BM), lambda i: (0, 0, i)),
            pl.BlockSpec((1, BM), lambda i: (0, i)),
            pl.BlockSpec((F, 1), lambda i: (0, 0)),
            pl.BlockSpec((F, NCLS), lambda i: (0, 0)),
        ],
        out_specs=pl.BlockSpec((F8, BM), lambda i: (0, i)),
        out_shape=jax.ShapeDtypeStruct((F8, NP), jnp.float32),
    )(u1T, p1, dinvT, b1c, W2)


def _tc3(u2T, p2, dinvT, b2c):
    def body(u_ref, p_ref, dinv_ref, b_ref, o_ref):
        tot = (u_ref[...] + p_ref[0] + p_ref[1] + p_ref[2] + p_ref[3])
        tot = tot * dinv_ref[...]
        z = tot[:NCLS, :] + b_ref[...]
        m = jnp.max(z, axis=0, keepdims=True)
        lse = jnp.log(jnp.sum(jnp.exp(z - m), axis=0, keepdims=True)) + m
        o_ref[...] = (z - lse).T

    return pl.pallas_call(
        body,
        grid=(NP // BM,),
        in_specs=[
            pl.BlockSpec((F8, BM), lambda i: (0, i)),
            pl.BlockSpec((4, F8, BM), lambda i: (0, 0, i)),
            pl.BlockSpec((1, BM), lambda i: (0, i)),
            pl.BlockSpec((NCLS, 1), lambda i: (0, 0)),
        ],
        out_specs=pl.BlockSpec((BM, NCLS), lambda i: (i, 0)),
        out_shape=jax.ShapeDtypeStruct((N, NCLS), jnp.float32),
    )(u2T, p2, dinvT, b2c)


def kernel(x, edge_index, W1, b1, W2, b2):
    src = edge_index[0]
    dst = edge_index[1]
    srcQ1 = src.reshape(NC, KE16 // NC, L)
    dstQ1 = dst.reshape(NC, KE16 // NC, L)
    srcQ2 = src.reshape(4, KE16 // 4, L)
    dstQ2 = dst.reshape(4, KE16 // 4, L)

    u1T = _tc1(x, W1)
    p1, dinvT = _agg16_deg(u1T, srcQ1, dstQ1)
    u2T = _tc2(u1T, p1, dinvT, b1.reshape(F, 1), W2)
    p2 = _agg8(u2T, srcQ2, dstQ2)
    return _tc3(u2T, p2, dinvT, b2.reshape(NCLS, 1))


# speedup vs baseline: 1.0432x; 1.0137x over previous
import functools

import jax
import jax.numpy as jnp
from jax import lax
from jax.experimental import pallas as pl
from jax.experimental.pallas import tpu as pltpu
from jax.experimental.pallas import tpu_sc as plsc

N = 10000
NP = 10240
E = 160000
D_IN = 256
F = 16
NCLS = 7

NC = 2
NS = 16
L = 16
NW = NC * NS
RPT = NP // NS
BM = 1024


def _sc_mesh():
    return plsc.VectorSubcoreMesh(core_axis_name="c", subcore_axis_name="s",
                                  num_cores=NC, num_subcores=NS)


_SC_PARAMS = pltpu.CompilerParams(needs_layout_passes=False,
                                  use_tc_tiling_on_sc=False)


KE16 = E // L


def _make_aggregate(FW, CHG):
    G = NS // FW
    SPLIT = NC * G
    KEQ = KE16 // SPLIT
    NCH = KEQ // CHG
    assert KEQ % CHG == 0 and NS % FW == 0 and KE16 % SPLIT == 0

    @functools.partial(
        pl.kernel,
        mesh=_sc_mesh(),
        compiler_params=_SC_PARAMS,
        out_type=jax.ShapeDtypeStruct((SPLIT, FW, NP), jnp.float32),
        scratch_types=[
            pltpu.VMEM((2, CHG, L), jnp.int32),
            pltpu.VMEM((2, CHG, L), jnp.int32),
            pltpu.VMEM((NP,), jnp.float32),
            pltpu.VMEM((NP,), jnp.float32),
            pltpu.SemaphoreType.DMA,
            pltpu.SemaphoreType.DMA,
            pltpu.SemaphoreType.DMA,
            pltpu.SemaphoreType.DMA,
        ],
    )
    def k(u_hbm, src_hbm, dst_hbm, out_hbm, src_v, dst_v, u_v, acc_v,
          sem_s0, sem_s1, sem_d0, sem_d1):
        c = lax.axis_index("c")
        s = lax.axis_index("s")
        col = s % FW
        q = c * G + s // FW

        ssems = (sem_s0, sem_s1)
        dsems = (sem_d0, sem_d1)

        def start(t, slot):
            pltpu.async_copy(src_hbm.at[q, pl.ds(t * CHG, CHG)],
                             src_v.at[slot], ssems[slot])
            pltpu.async_copy(dst_hbm.at[q, pl.ds(t * CHG, CHG)],
                             dst_v.at[slot], dsems[slot])

        def wait(t, slot):
            pltpu.make_async_copy(src_hbm.at[q, pl.ds(t * CHG, CHG)],
                                  src_v.at[slot], ssems[slot]).wait()
            pltpu.make_async_copy(dst_hbm.at[q, pl.ds(t * CHG, CHG)],
                                  dst_v.at[slot], dsems[slot]).wait()

        start(0, 0)
        pltpu.sync_copy(u_hbm.at[col], u_v)

        z16 = jnp.zeros((L,), jnp.float32)

        @plsc.parallel_loop(0, NP // L, unroll=4)
        def _(i):
            acc_v[pl.ds(i * L, L)] = z16

        for t in range(NCH):
            slot = t % 2
            if t + 1 < NCH:
                start(t + 1, (t + 1) % 2)
            wait(t, slot)

            @plsc.parallel_loop(0, CHG, unroll=4)
            def _(gf):
                sidx = src_v[slot, gf, :]
                didx = dst_v[slot, gf, :]
                vals = plsc.load_gather(u_v, [sidx])
                plsc.addupdate_scatter(acc_v, [didx], vals)

        pltpu.sync_copy(acc_v, out_hbm.at[q, col])

    return k


_agg8 = _make_aggregate(8, 500)

GPC = KE16 // NS
CH1 = 500
NCH1 = (KE16 // NC) // CH1


def _agg16_deg(uT, src2, dst2):
    @functools.partial(
        pl.kernel,
        mesh=_sc_mesh(),
        compiler_params=_SC_PARAMS,
        out_type=[
            jax.ShapeDtypeStruct((NC, F, NP), jnp.float32),
            jax.ShapeDtypeStruct((1, NP), jnp.float32),
        ],
        scratch_types=[
            pltpu.VMEM((2, CH1, L), jnp.int32),
            pltpu.VMEM((2, CH1, L), jnp.int32),
            pltpu.VMEM((GPC, L), jnp.int32),
            pltpu.VMEM((NP,), jnp.float32),
            pltpu.VMEM((NS, RPT), jnp.float32),
            pltpu.VMEM((NP,), jnp.float32),
            pltpu.VMEM((NP,), jnp.float32),
            pltpu.VMEM((NP,), jnp.float32),
            pltpu.VMEM_SHARED((NS, NP), jnp.float32),
            pltpu.VMEM_SHARED((NP,), jnp.float32),
            pltpu.SemaphoreType.DMA,
            pltpu.SemaphoreType.DMA,
            pltpu.SemaphoreType.DMA,
            pltpu.SemaphoreType.DMA,
            pltpu.SemaphoreType.DMA,
            pltpu.SemaphoreType.DMA,
        ],
    )
    def k(u_hbm, src_hbm, dst_hbm, out_hbm, dinv_hbm,
          src_v, dst_v, hist_v, deg_v, red_v, dinv_v, u_v, acc_v,
          deg_sh, dinv_sh, sem_s0, sem_s1, sem_d0, sem_d1, sem_h, sem_u):
        c = lax.axis_index("c")
        s = lax.axis_index("s")

        ssems = (sem_s0, sem_s1)
        dsems = (sem_d0, sem_d1)

        def start(t, slot):
            pltpu.async_copy(src_hbm.at[c, pl.ds(t * CH1, CH1)],
                             src_v.at[slot], ssems[slot])
            pltpu.async_copy(dst_hbm.at[c, pl.ds(t * CH1, CH1)],
                             dst_v.at[slot], dsems[slot])

        def wait(t, slot):
            pltpu.make_async_copy(src_hbm.at[c, pl.ds(t * CH1, CH1)],
                                  src_v.at[slot], ssems[slot]).wait()
            pltpu.make_async_copy(dst_hbm.at[c, pl.ds(t * CH1, CH1)],
                                  dst_v.at[slot], dsems[slot]).wait()

        hc = s // (NS // NC)
        hr = s % (NS // NC)
        pltpu.async_copy(dst_hbm.at[hc, pl.ds(hr * GPC, GPC)], hist_v, sem_h)
        pltpu.async_copy(u_hbm.at[s], u_v, sem_u)
        start(0, 0)

        z16 = jnp.zeros((L,), jnp.float32)

        @plsc.parallel_loop(0, NP // L, unroll=4)
        def _(i):
            deg_v[pl.ds(i * L, L)] = z16
            acc_v[pl.ds(i * L, L)] = z16

        pltpu.make_async_copy(dst_hbm.at[hc, pl.ds(hr * GPC, GPC)],
                              hist_v, sem_h).wait()
        ones16 = jnp.ones((L,), jnp.float32)

        @plsc.parallel_loop(0, GPC, unroll=4)
        def _(g):
            plsc.addupdate_scatter(deg_v, [hist_v[g, :]], ones16)

        pltpu.sync_copy(deg_v, deg_sh.at[s])
        plsc.subcore_barrier()

        for r in range(NS):
            pltpu.async_copy(deg_sh.at[r, pl.ds(s * RPT, RPT)],
                             red_v.at[r], sem_h)
        for r in range(NS):
            pltpu.make_async_copy(deg_sh.at[r, pl.ds(s * RPT, RPT)],
                                  red_v.at[r], sem_h).wait()

        half = jnp.full((L,), 0.5, jnp.float32)
        three_half = jnp.full((L,), 1.5, jnp.float32)
        magic = jnp.full((L,), 0x5F3759DF, jnp.int32)

        @plsc.parallel_loop(0, RPT // L, unroll=2)
        def _(t):
            d = red_v[0, pl.ds(t * L, L)]
            for r in range(1, NS):
                d = d + red_v[r, pl.ds(t * L, L)]
            d = d + 1.0
            yi = magic - lax.shift_right_arithmetic(plsc.bitcast(d, jnp.int32),
                                                    jnp.full((L,), 1, jnp.int32))
            y = plsc.bitcast(yi, jnp.float32)
            hd = d * half
            for _ in range(3):
                y = y * (three_half - hd * y * y)
            deg_v[pl.ds(t * L, L)] = y

        pltpu.sync_copy(deg_v.at[pl.ds(0, RPT)],
                        dinv_sh.at[pl.ds(s * RPT, RPT)])

        @pl.when(c == 0)
        def _():
            pltpu.sync_copy(deg_v.at[pl.ds(0, RPT)],
                            dinv_hbm.at[0, pl.ds(s * RPT, RPT)])

        plsc.subcore_barrier()
        pltpu.sync_copy(dinv_sh, dinv_v)
        pltpu.make_async_copy(u_hbm.at[s], u_v, sem_u).wait()

        @plsc.parallel_loop(0, NP // L, unroll=4)
        def _(i):
            u_v[pl.ds(i * L, L)] = u_v[pl.ds(i * L, L)] * dinv_v[pl.ds(i * L, L)]

        for t in range(NCH1):
            slot = t % 2
            if t + 1 < NCH1:
                start(t + 1, (t + 1) % 2)
            wait(t, slot)

            @plsc.parallel_loop(0, CH1, unroll=4)
            def _(gf):
                sidx = src_v[slot, gf, :]
                didx = dst_v[slot, gf, :]
                vals = plsc.load_gather(u_v, [sidx])
                plsc.addupdate_scatter(acc_v, [didx], vals)

        pltpu.sync_copy(acc_v, out_hbm.at[c, s])

    return k(uT, src2, dst2)


def _tc1(x, W1):
    def body(x_ref, w_ref, u_ref):
        u_ref[...] = lax.dot_general(w_ref[...], x_ref[...],
                                     (((0,), (1,)), ((), ())),
                                     preferred_element_type=jnp.float32)

    return pl.pallas_call(
        body,
        grid=(NP // BM,),
        in_specs=[
            pl.BlockSpec((BM, D_IN), lambda i: (i, 0)),
            pl.BlockSpec((D_IN, F), lambda i: (0, 0)),
        ],
        out_specs=pl.BlockSpec((F, BM), lambda i: (0, i)),
        out_shape=jax.ShapeDtypeStruct((F, NP), jnp.float32),
    )(x, W1)


F8 = 8


def _tc2(u1T, p1, dinvT, b1c, W2):
    def body(u_ref, p_ref, dinv_ref, b_ref, w_ref, u2_ref):
        tot = u_ref[...] * dinv_ref[...] + p_ref[0] + p_ref[1]
        h = jnp.maximum(tot * dinv_ref[...] + b_ref[...], 0.0)
        u2 = lax.dot_general(w_ref[...], h, (((0,), (0,)), ((), ())),
                             preferred_element_type=jnp.float32)
        u2_ref[0:NCLS, :] = u2 * dinv_ref[...]
        u2_ref[NCLS:F8, :] = jnp.zeros((F8 - NCLS, BM), jnp.float32)

    return pl.pallas_call(
        body,
        grid=(NP // BM,),
        in_specs=[
            pl.BlockSpec((F, BM), lambda i: (0, i)),
            pl.BlockSpec((NC, F, BM), lambda i: (0, 0, i)),
            pl.BlockSpec((1, BM), lambda i: (0, i)),
            pl.BlockSpec((F, 1), lambda i: (0, 0)),
            pl.BlockSpec((F, NCLS), lambda i: (0, 0)),
        ],
        out_specs=pl.BlockSpec((F8, BM), lambda i: (0, i)),
        out_shape=jax.ShapeDtypeStruct((F8, NP), jnp.float32),
    )(u1T, p1, dinvT, b1c, W2)


def _tc3(u2T, p2, dinvT, b2c):
    def body(u_ref, p_ref, dinv_ref, b_ref, o_ref):
        tot = (u_ref[...] + p_ref[0] + p_ref[1] + p_ref[2] + p_ref[3])
        tot = tot * dinv_ref[...]
        z = tot[:NCLS, :] + b_ref[...]
        m = jnp.max(z, axis=0, keepdims=True)
        lse = jnp.log(jnp.sum(jnp.exp(z - m), axis=0, keepdims=True)) + m
        o_ref[...] = (z - lse).T

    return pl.pallas_call(
        body,
        grid=(NP // BM,),
        in_specs=[
            pl.BlockSpec((F8, BM), lambda i: (0, i)),
            pl.BlockSpec((4, F8, BM), lambda i: (0, 0, i)),
            pl.BlockSpec((1, BM), lambda i: (0, i)),
            pl.BlockSpec((NCLS, 1), lambda i: (0, 0)),
        ],
        out_specs=pl.BlockSpec((BM, NCLS), lambda i: (i, 0)),
        out_shape=jax.ShapeDtypeStruct((N, NCLS), jnp.float32),
    )(u2T, p2, dinvT, b2c)


def kernel(x, edge_index, W1, b1, W2, b2):
    src = edge_index[0]
    dst = edge_index[1]
    srcQ1 = src.reshape(NC, KE16 // NC, L)
    dstQ1 = dst.reshape(NC, KE16 // NC, L)
    srcQ2 = src.reshape(4, KE16 // 4, L)
    dstQ2 = dst.reshape(4, KE16 // 4, L)

    u1T = _tc1(x, W1)
    p1, dinvT = _agg16_deg(u1T, srcQ1, dstQ1)
    u2T = _tc2(u1T, p1, dinvT, b1.reshape(F, 1), W2)
    p2 = _agg8(u2T, srcQ2, dstQ2)
    return _tc3(u2T, p2, dinvT, b2.reshape(NCLS, 1))
